# baseline (speedup 1.0000x reference)
import functools

import jax
import jax.numpy as jnp
from jax.experimental import pallas as pl
from jax.experimental.pallas import tpu as pltpu

MARGIN_ = 1.0


def _triplet_block(e_blk_ref, e_full_ref, t_ref, loss_ref, cnt_ref, acc_ref,
                   efa_ref, sqf_ref, lvec_ref, cvec_ref, avec_ref,
                   *, blk: int, batch: int, cblk: int, dim: int):
    i = pl.program_id(0)

    @pl.when(i == 0)
    def _prep():
        ef = e_full_ref[...]
        efa_ref[...] = -2.0 * ef
        sqf_ref[...] = jnp.sum(ef * ef, axis=1).reshape(1, batch)

    eb = e_blk_ref[...]
    sq_b = jnp.sum(eb * eb, axis=1)

    gram2 = jax.lax.dot_general(
        eb, efa_ref[...], (((1,), (1,)), ((), ())),
        preferred_element_type=jnp.float32)
    sp = sqf_ref[...] + gram2

    t_full = t_ref[0, :]
    t_blk = t_ref[0, pl.ds(i * blk, blk)]
    same = t_blk[:, None] == t_full[None, :]

    neg_sp = jnp.where(same, jnp.inf, sp)
    min_sp = jnp.maximum(jnp.min(neg_sp, axis=1), -sq_b)

    min_d = jnp.sqrt(jnp.maximum(min_sp + sq_b, 0.0))
    thr = (jnp.where(min_d >= MARGIN_,
                     (min_d - MARGIN_) ** 2,
                     -1.0) - sq_b)[:, None]
    ushift = (MARGIN_ - min_sp)[:, None]
    minsp_col = min_sp[:, None]

    @pl.when(i == 0)
    def _init():
        lvec_ref[...] = jnp.zeros((1, cblk), jnp.float32)
        cvec_ref[...] = jnp.zeros((1, cblk), jnp.float32)
        avec_ref[...] = jnp.zeros((1, cblk), jnp.float32)

    ratio = blk // cblk
    rows = i * blk + jax.lax.broadcasted_iota(jnp.int32, (blk, 1), 0)
    ones_row = jnp.ones((1, blk), jnp.float32)
    ones_b16 = jnp.ones((1, blk), jnp.bfloat16)

    def chunk_sums(c, need_upper):
        sl = slice(c * cblk, (c + 1) * cblk)
        spc = sp[:, sl]
        tri = same[:, sl] & (spc > thr)
        if need_upper:
            cols = c * cblk + jax.lax.broadcasted_iota(jnp.int32, (1, cblk), 1)
            tri = tri & (rows < cols)
        trib = tri.astype(jnp.bfloat16)
        accb = jnp.where(spc < minsp_col, trib, jnp.bfloat16(0))
        lossm = jnp.where(tri, jnp.maximum(spc + ushift, 0.0), 0.0)
        dot = lambda o, x: jax.lax.dot_general(
            o, x, (((1,), (0,)), ((), ())),
            preferred_element_type=jnp.float32)
        lvec_ref[...] += dot(ones_row, lossm)
        cvec_ref[...] += dot(ones_b16, trib)
        avec_ref[...] += dot(ones_b16, accb)

    for c in range(batch // cblk):
        @pl.when(c >= (i + 1) * ratio)
        def _full(c=c):
            chunk_sums(c, need_upper=False)

        @pl.when((c >= i * ratio) & (c < (i + 1) * ratio))
        def _diag(c=c):
            chunk_sums(c, need_upper=True)

    @pl.when(i == pl.num_programs(0) - 1)
    def _finish():
        loss_ref[...] = jnp.sum(lvec_ref[...]).reshape(1, 1)
        cnt_ref[...] = jnp.sum(cvec_ref[...]).reshape(1, 1)
        acc_ref[...] = jnp.sum(avec_ref[...]).reshape(1, 1)


@jax.jit
def kernel(embeddings, targets):
    batch, dim = embeddings.shape
    blk = 1024
    cblk = 512
    t32 = targets.astype(jnp.int32).reshape(1, batch)

    loss_sum, cnt, acc_sum = pl.pallas_call(
        functools.partial(_triplet_block, blk=blk, batch=batch, cblk=cblk,
                          dim=dim),
        grid=(batch // blk,),
        in_specs=[
            pl.BlockSpec((blk, dim), lambda i: (i, 0)),
            pl.BlockSpec((batch, dim), lambda i: (0, 0)),
            pl.BlockSpec((1, batch), lambda i: (0, 0)),
        ],
        out_specs=[
            pl.BlockSpec((1, 1), lambda i: (0, 0)),
            pl.BlockSpec((1, 1), lambda i: (0, 0)),
            pl.BlockSpec((1, 1), lambda i: (0, 0)),
        ],
        out_shape=[
            jax.ShapeDtypeStruct((1, 1), jnp.float32),
            jax.ShapeDtypeStruct((1, 1), jnp.float32),
            jax.ShapeDtypeStruct((1, 1), jnp.float32),
        ],
        scratch_shapes=[
            pltpu.VMEM((batch, dim), jnp.float32),
            pltpu.VMEM((1, batch), jnp.float32),
            pltpu.VMEM((1, cblk), jnp.float32),
            pltpu.VMEM((1, cblk), jnp.float32),
            pltpu.VMEM((1, cblk), jnp.float32),
        ],
    )(embeddings, embeddings, t32)

    loss = loss_sum[0, 0] / cnt[0, 0]
    accuracy = acc_sum[0, 0] / cnt[0, 0]
    return (loss, accuracy)


# speedup vs baseline: 1.0399x; 1.0399x over previous
import functools

import jax
import jax.numpy as jnp
from jax.experimental import pallas as pl
from jax.experimental.pallas import tpu as pltpu

MARGIN_ = 1.0


def _triplet_block(e_blk_ref, e_full_ref, t_ref, loss_ref, cnt_ref, acc_ref,
                   efa_ref, sqf_ref, lvec_ref, cvec_ref, avec_ref,
                   *, blk: int, batch: int, cblk: int, dim: int):
    i = pl.program_id(0)

    @pl.when(i == 0)
    def _prep():
        ef = e_full_ref[...]
        efa_ref[...] = -2.0 * ef
        sqf_ref[...] = jnp.sum(ef * ef, axis=1).reshape(1, batch)

    eb = e_blk_ref[...]
    sq_b = jnp.sum(eb * eb, axis=1)

    gram2 = jax.lax.dot_general(
        eb, efa_ref[...], (((1,), (1,)), ((), ())),
        preferred_element_type=jnp.float32)
    sp = sqf_ref[...] + gram2

    t_full = t_ref[0, :]
    t_blk = t_ref[0, pl.ds(i * blk, blk)]
    same = t_blk[:, None] == t_full[None, :]

    neg_sp = jnp.where(same, jnp.inf, sp)
    min_sp = jnp.maximum(jnp.min(neg_sp, axis=1), -sq_b)

    min_d = jnp.sqrt(jnp.maximum(min_sp + sq_b, 0.0))
    thr = (jnp.where(min_d >= MARGIN_,
                     (min_d - MARGIN_) ** 2,
                     -1.0) - sq_b)[:, None]
    ushift = (MARGIN_ - min_sp)[:, None]
    minsp_col = min_sp[:, None]

    @pl.when(i == 0)
    def _init():
        lvec_ref[...] = jnp.zeros((1, cblk), jnp.float32)
        cvec_ref[...] = jnp.zeros((1, cblk), jnp.float32)
        avec_ref[...] = jnp.zeros((1, cblk), jnp.float32)

    ratio = blk // cblk
    rows = i * blk + jax.lax.broadcasted_iota(jnp.int32, (blk, 1), 0)
    ones_row = jnp.ones((1, blk), jnp.float32)
    ones_b16 = jnp.ones((1, blk), jnp.bfloat16)

    def chunk_sums(c, need_upper):
        sl = slice(c * cblk, (c + 1) * cblk)
        spc = sp[:, sl]
        tri = same[:, sl] & (spc > thr)
        if need_upper:
            cols = c * cblk + jax.lax.broadcasted_iota(jnp.int32, (1, cblk), 1)
            tri = tri & (rows < cols)
        trib = tri.astype(jnp.bfloat16)
        accb = jnp.where(spc < minsp_col, trib, jnp.bfloat16(0))
        lossm = jnp.where(tri, jnp.maximum(spc + ushift, 0.0), 0.0)
        dot = lambda o, x: jax.lax.dot_general(
            o, x, (((1,), (0,)), ((), ())),
            preferred_element_type=jnp.float32)
        lvec_ref[...] += dot(ones_row, lossm)
        cvec_ref[...] += dot(ones_b16, trib)
        avec_ref[...] += dot(ones_b16, accb)

    for c in range(batch // cblk):
        @pl.when(c >= (i + 1) * ratio)
        def _full(c=c):
            chunk_sums(c, need_upper=False)

        @pl.when((c >= i * ratio) & (c < (i + 1) * ratio))
        def _diag(c=c):
            chunk_sums(c, need_upper=True)

    @pl.when(i == pl.num_programs(0) - 1)
    def _finish():
        loss_ref[...] = jnp.sum(lvec_ref[...]).reshape(1, 1)
        cnt_ref[...] = jnp.sum(cvec_ref[...]).reshape(1, 1)
        acc_ref[...] = jnp.sum(avec_ref[...]).reshape(1, 1)


@jax.jit
def kernel(embeddings, targets):
    batch, dim = embeddings.shape
    blk = 512
    cblk = 512
    t32 = targets.astype(jnp.int32).reshape(1, batch)

    loss_sum, cnt, acc_sum = pl.pallas_call(
        functools.partial(_triplet_block, blk=blk, batch=batch, cblk=cblk,
                          dim=dim),
        grid=(batch // blk,),
        in_specs=[
            pl.BlockSpec((blk, dim), lambda i: (i, 0)),
            pl.BlockSpec((batch, dim), lambda i: (0, 0)),
            pl.BlockSpec((1, batch), lambda i: (0, 0)),
        ],
        out_specs=[
            pl.BlockSpec((1, 1), lambda i: (0, 0)),
            pl.BlockSpec((1, 1), lambda i: (0, 0)),
            pl.BlockSpec((1, 1), lambda i: (0, 0)),
        ],
        out_shape=[
            jax.ShapeDtypeStruct((1, 1), jnp.float32),
            jax.ShapeDtypeStruct((1, 1), jnp.float32),
            jax.ShapeDtypeStruct((1, 1), jnp.float32),
        ],
        scratch_shapes=[
            pltpu.VMEM((batch, dim), jnp.float32),
            pltpu.VMEM((1, batch), jnp.float32),
            pltpu.VMEM((1, cblk), jnp.float32),
            pltpu.VMEM((1, cblk), jnp.float32),
            pltpu.VMEM((1, cblk), jnp.float32),
        ],
    )(embeddings, embeddings, t32)

    loss = loss_sum[0, 0] / cnt[0, 0]
    accuracy = acc_sum[0, 0] / cnt[0, 0]
    return (loss, accuracy)
